# Initial kernel scaffold; baseline (speedup 1.0000x reference)
import jax
import jax.numpy as jnp
from jax.experimental import pallas as pl


def kernel(Z, R1, senders, receivers, params):
    raise NotImplementedError("write your pallas kernel here")


# speedup vs baseline: 2.4865x; 2.4865x over previous
import jax
import jax.numpy as jnp
from jax import lax
from jax.experimental import pallas as pl
from jax.experimental.pallas import tpu as pltpu
from jax.experimental.pallas import tpu_sc as plsc

N = 10000
NPAD = 10240
E = 320000
NODE = 128
NB = 16
NC9 = 72
CP = 128
CUTOFF = 5.0
P = 6
KEPS = 37.27404695554026

SC_CORES = 2
SC_TILES = 16
NW = SC_CORES * SC_TILES
CH = 80
ECH = E // NW // CH
ZCH = NPAD // NW // CH

def _mesh():
    return plsc.VectorSubcoreMesh(core_axis_name="c", subcore_axis_name="s")


def _sc_gather(table, idxs, D, nch, nbuf):
    n = len(idxs)
    perw = nch * CH
    ngroups = nch // nbuf
    out_type = tuple(jax.ShapeDtypeStruct((NW * perw, D), jnp.float32)
                     for _ in range(n))

    def body(table_ref, *rest):
        idx_refs = rest[:n]
        out_refs = rest[n:2 * n]
        idx_v, rows_v, semg, semw = rest[2 * n:]
        w = lax.axis_index("s") * SC_CORES + lax.axis_index("c")
        base = w * perw
        for t in range(n):
            pltpu.sync_copy(idx_refs[t].at[w], idx_v)

            def group(g, _):
                hg = []
                for b in range(nbuf):
                    j = g * nbuf + b
                    hg.append(pltpu.async_copy(
                        table_ref.at[idx_v.at[j]], rows_v.at[b], semg))
                hw = []
                for b in range(nbuf):
                    j = g * nbuf + b
                    hg[b].wait()
                    hw.append(pltpu.async_copy(
                        rows_v.at[b],
                        out_refs[t].at[pl.ds(base + j * CH, CH)], semw))
                for h in hw:
                    h.wait()
                return 0

            lax.fori_loop(0, ngroups, group, 0)

    f = pl.kernel(
        body, out_type=out_type, mesh=_mesh(),
        scratch_types=[
            pltpu.VMEM((nch, CH), jnp.int32),
            pltpu.VMEM((nbuf, CH, D), jnp.float32),
            pltpu.SemaphoreType.DMA,
            pltpu.SemaphoreType.DMA,
        ])
    return list(f(table, *idxs))


def _sc_scatter(vals, ridx, C, nbuf=2):
    ngroups = ECH // nbuf
    ntail = ECH % nbuf
    rpt = NPAD // SC_TILES
    zeros = jnp.zeros((CH, C), jnp.float32)
    out_type = jax.ShapeDtypeStruct((2 * NPAD, C), jnp.float32)

    def body(vals_ref, ridx_ref, z_ref, out01, idx_v, rows_v, acc,
             semg, sems):
        c = lax.axis_index("c")
        s = lax.axis_index("s")
        w = s * SC_CORES + c
        pltpu.sync_copy(z_ref, rows_v.at[0])
        for q in range(rpt // CH):
            pltpu.sync_copy(rows_v.at[0], acc.at[pl.ds(s * rpt + q * CH, CH)])
        pltpu.sync_copy(ridx_ref.at[w], idx_v)
        plsc.subcore_barrier()
        base = w * ECH * CH

        def group(g, _):
            hg = []
            for b in range(nbuf):
                j = g * nbuf + b
                hg.append(pltpu.async_copy(
                    vals_ref.at[pl.ds(base + j * CH, CH)], rows_v.at[b], semg))
            hs = []
            for b in range(nbuf):
                j = g * nbuf + b
                hg[b].wait()
                hs.append(pltpu.async_copy(
                    rows_v.at[b], acc.at[idx_v.at[j]], sems, add=True))
            for h in hs:
                h.wait()
            return 0

        lax.fori_loop(0, ngroups, group, 0)
        for t in range(ntail):
            j = ngroups * nbuf + t
            pltpu.sync_copy(vals_ref.at[pl.ds(base + j * CH, CH)], rows_v.at[0])
            pltpu.async_copy(rows_v.at[0], acc.at[idx_v.at[j]], sems,
                             add=True).wait()
        plsc.subcore_barrier()
        obase = c * NPAD + s * rpt
        for q in range(rpt // CH):
            pltpu.sync_copy(acc.at[pl.ds(s * rpt + q * CH, CH)],
                            rows_v.at[q % nbuf])
            pltpu.sync_copy(rows_v.at[q % nbuf],
                            out01.at[pl.ds(obase + q * CH, CH)])

    f = pl.kernel(
        body, out_type=out_type, mesh=_mesh(),
        scratch_types=[
            pltpu.VMEM((ECH, CH), jnp.int32),
            pltpu.VMEM((nbuf, CH, C), jnp.float32),
            pltpu.VMEM_SHARED((NPAD, C), jnp.float32),
            pltpu.SemaphoreType.DMA,
            pltpu.SemaphoreType.DMA,
        ])
    return f(vals, ridx, zeros)


def _silu(x):
    return x * jax.nn.sigmoid(x)


def _wspec(a):
    return pl.BlockSpec(a.shape, lambda i: (0,) * a.ndim)


def _xspec(be, d):
    return pl.BlockSpec((be, d), lambda i: (i, 0))


def _envelope(r):
    x = jnp.clip(r * (1.0 / CUTOFF), 0.0, 1.0)
    x6 = x * x * x * x * x * x
    env = 1.0 - 28.0 * x6 + 48.0 * x6 * x - 21.0 * x6 * x * x
    return jnp.where(r < CUTOFF, env, 0.0)


def _edge_call(r1, fi, fj, wr, wfi, wfj, b1, w2, b2, be=1280):
    grid = (E // be,)

    def body(r_ref, fi_ref, fj_ref, wr_r, wfi_r, wfj_r, b1_r, w2_r, b2_r, out):
        r = r_ref[...]
        rs = jnp.maximum(r, 1e-2)
        freq = (lax.broadcasted_iota(jnp.int32, (1, NB), 1) + 1
                ).astype(jnp.float32) * (jnp.pi / CUTOFF)
        rbf = jnp.sin(freq * rs) / rs
        h = (jnp.dot(rbf, wr_r[...], preferred_element_type=jnp.float32)
             + jnp.dot(fi_ref[...], wfi_r[...], preferred_element_type=jnp.float32)
             + jnp.dot(fj_ref[...], wfj_r[...], preferred_element_type=jnp.float32)
             + b1_r[...])
        h = _silu(h)
        out[...] = jnp.dot(h, w2_r[...], preferred_element_type=jnp.float32) \
            + b2_r[...]

    return pl.pallas_call(
        body,
        grid=grid,
        in_specs=[_xspec(be, 1), _xspec(be, NODE), _xspec(be, NODE),
                  _wspec(wr), _wspec(wfi), _wspec(wfj), _wspec(b1),
                  _wspec(w2), _wspec(b2)],
        out_specs=_xspec(be, NODE),
        out_shape=jax.ShapeDtypeStruct((E, NODE), jnp.float32),
    )(r1, fi, fj, wr, wfi, wfj, b1, w2, b2)


def _eqmsg_call(r1, fi, fj, pr, ps, ed, ws, dout, be=1280):
    grid = (E // be,)
    has_an = pr is not None
    wfi, wfj, wan, wed, b1, w2, b2, w3, b3 = ws

    def body(*refs):
        if has_an:
            (r_ref, fi_ref, fj_ref, pr_ref, ps_ref, ed_ref,
             wfi_r, wfj_r, wan_r, wed_r, b1_r, w2_r, b2_r, w3_r, b3_r,
             out) = refs
        else:
            (r_ref, fi_ref, fj_ref, ed_ref,
             wfi_r, wfj_r, wed_r, b1_r, w2_r, b2_r, w3_r, b3_r, out) = refs
        h = (jnp.dot(fi_ref[...], wfi_r[...], preferred_element_type=jnp.float32)
             + jnp.dot(fj_ref[...], wfj_r[...], preferred_element_type=jnp.float32)
             + jnp.dot(ed_ref[...], wed_r[...], preferred_element_type=jnp.float32)
             + b1_r[...])
        if has_an:
            h = h + jnp.dot(pr_ref[...] - ps_ref[...], wan_r[...],
                            preferred_element_type=jnp.float32)
        h = _silu(h)
        h = _silu(jnp.dot(h, w2_r[...], preferred_element_type=jnp.float32)
                  + b2_r[...])
        o = jnp.dot(h, w3_r[...], preferred_element_type=jnp.float32) + b3_r[...]
        out[...] = o * _envelope(r_ref[...])

    xs = [r1, fi, fj] + ([pr, ps] if has_an else []) + [ed]
    wlist = [wfi, wfj] + ([wan] if has_an else []) + [wed, b1, w2, b2, w3, b3]
    in_specs = ([_xspec(be, 1), _xspec(be, NODE), _xspec(be, NODE)]
                + ([_xspec(be, CP), _xspec(be, CP)] if has_an else [])
                + [_xspec(be, NODE)]
                + [_wspec(w) for w in wlist])
    return pl.pallas_call(
        body,
        grid=grid,
        in_specs=in_specs,
        out_specs=_xspec(be, dout),
        out_shape=jax.ShapeDtypeStruct((E, dout), jnp.float32),
    )(*(xs + wlist))


def _halfspec(be, d):
    return pl.BlockSpec((be, d), lambda i: (i + NPAD // be, 0))


def _combine_call(p01, d, be=1024):
    grid = (NPAD // be,)

    def body(a, b, out):
        out[...] = a[...] + b[...]

    return pl.pallas_call(
        body, grid=grid,
        in_specs=[_xspec(be, d), _halfspec(be, d)],
        out_specs=_xspec(be, d),
        out_shape=jax.ShapeDtypeStruct((NPAD, d), jnp.float32),
    )(p01, p01)


def _upd_call(nodes, a01, ws, be=1024):
    grid = (NPAD // be,)
    wn, wa, b1, w2, b2, w3, b3 = ws

    def body(n_ref, a0_ref, a1_ref, wn_r, wa_r, b1_r, w2_r, b2_r, w3_r, b3_r,
             out):
        nd = n_ref[...]
        agg = a0_ref[...] + a1_ref[...]
        h = (jnp.dot(nd, wn_r[...], preferred_element_type=jnp.float32)
             + jnp.dot(agg, wa_r[...], preferred_element_type=jnp.float32)
             + b1_r[...])
        h = _silu(h)
        h = _silu(jnp.dot(h, w2_r[...], preferred_element_type=jnp.float32)
                  + b2_r[...])
        out[...] = nd + jnp.dot(h, w3_r[...],
                                preferred_element_type=jnp.float32) + b3_r[...]

    return pl.pallas_call(
        body, grid=grid,
        in_specs=[_xspec(be, NODE), _xspec(be, NODE), _halfspec(be, NODE)]
        + [_wspec(w) for w in ws],
        out_specs=_xspec(be, NODE),
        out_shape=jax.ShapeDtypeStruct((NPAD, NODE), jnp.float32),
    )(nodes, a01, a01, *ws)


def _mono_call(nodes, w1, b1, w2, b2, be=1024):
    grid = (NPAD // be,)

    def body(n_ref, w1_r, b1_r, w2_r, b2_r, out):
        h = _silu(jnp.dot(n_ref[...], w1_r[...],
                          preferred_element_type=jnp.float32) + b1_r[...])
        out[...] = (jnp.dot(h, w2_r[...], preferred_element_type=jnp.float32)
                    + b2_r[...]) * (1.0 / KEPS)

    return pl.pallas_call(
        body, grid=grid,
        in_specs=[_xspec(be, NODE), _wspec(w1), _wspec(b1), _wspec(w2),
                  _wspec(b2)],
        out_specs=_xspec(be, 1),
        out_shape=jax.ShapeDtypeStruct((NPAD, 1), jnp.float32),
    )(nodes, w1, b1, w2, b2)


def _row(b):
    return b.reshape(1, -1)


def _pad_cols(w, width):
    return jnp.pad(w, ((0, 0), (0, width - w.shape[1])))


def _pad_rows(w, height):
    return jnp.pad(w, ((0, height - w.shape[0]), (0, 0)))


def kernel(Z, R1, senders, receivers, params):
    Zp = jnp.pad(Z.astype(jnp.int32), (0, NPAD - N)).reshape(NW, ZCH, CH)
    send_r = senders.astype(jnp.int32).reshape(NW, ECH, CH)
    recv_r = receivers.astype(jnp.int32).reshape(NW, ECH, CH)
    r1c = R1.reshape(E, 1)

    nodes = _sc_gather(params["emb"], [Zp], D=NODE, nch=ZCH, nbuf=ZCH)[0]

    fi, fj = _sc_gather(nodes, [send_r, recv_r], D=NODE, nch=ECH, nbuf=5)

    (w1e, b1e), (w2e, b2e) = params["edge"]
    edges = _edge_call(r1c, fi, fj,
                       w1e[:NB], w1e[NB:NB + NODE], w1e[NB + NODE:],
                       _row(b1e), w2e, _row(b2e))

    pr = ps = None
    for s, lp in enumerate(params["steps"]):
        (w1, b1), (w2, b2), (w3, b3) = lp["eq"]
        if s == 0:
            ws = (w1[:NODE], w1[NODE:2 * NODE], None, w1[2 * NODE:],
                  _row(b1), w2, _row(b2), _pad_cols(w3, CP),
                  _row(jnp.pad(b3, (0, CP - NC9))))
        else:
            ws = (w1[:NODE], w1[NODE:2 * NODE],
                  _pad_rows(w1[2 * NODE:2 * NODE + NC9], CP),
                  w1[2 * NODE + NC9:],
                  _row(b1), w2, _row(b2), _pad_cols(w3, CP),
                  _row(jnp.pad(b3, (0, CP - NC9))))
        coeffs = _eqmsg_call(r1c, fi, fj, pr, ps, edges, ws, dout=CP)

        poles = _combine_call(_sc_scatter(coeffs, recv_r, C=CP), d=CP)

        pr, ps = _sc_gather(poles, [recv_r, send_r], D=CP, nch=ECH, nbuf=5)

        (w1, b1), (w2, b2), (w3, b3) = lp["msg"]
        ws = (w1[:NODE], w1[NODE:2 * NODE],
              _pad_rows(w1[2 * NODE:2 * NODE + NC9], CP),
              w1[2 * NODE + NC9:],
              _row(b1), w2, _row(b2), w3, _row(b3))
        msgs = _eqmsg_call(r1c, fi, fj, pr, ps, edges, ws, dout=NODE)

        a01 = _sc_scatter(msgs, recv_r, C=NODE)
        (w1, b1), (w2, b2), (w3, b3) = lp["upd"]
        nodes = _upd_call(nodes, a01,
                          (w1[:NODE], w1[NODE:], _row(b1), w2, _row(b2),
                           w3, _row(b3)))

        if s < len(params["steps"]) - 1:
            fi, fj = _sc_gather(nodes, [send_r, recv_r], D=NODE, nch=ECH,
                                nbuf=5)

    (w1m, b1m), (w2m, b2m) = params["mono"]
    monos = _mono_call(nodes, w1m, _row(b1m), w2m, _row(b2m))
    return jnp.concatenate([nodes[:N], monos[:N]], axis=-1)


# speedup vs baseline: 2.5300x; 1.0175x over previous
import jax
import jax.numpy as jnp
from jax import lax
from jax.experimental import pallas as pl
from jax.experimental.pallas import tpu as pltpu
from jax.experimental.pallas import tpu_sc as plsc

N = 10000
NPAD = 10240
E = 320000
NODE = 128
NB = 16
NC9 = 72
CP = 128
CUTOFF = 5.0
P = 6
KEPS = 37.27404695554026

SC_CORES = 2
SC_TILES = 16
NW = SC_CORES * SC_TILES
CH = 80
ECH = E // NW // CH
ZCH = NPAD // NW // CH

def _mesh():
    return plsc.VectorSubcoreMesh(core_axis_name="c", subcore_axis_name="s")


def _sc_gather(table, idxs, D, nch, nbuf, dtype=jnp.float32):
    n = len(idxs)
    perw = nch * CH
    ngroups = nch // nbuf
    out_type = tuple(jax.ShapeDtypeStruct((NW * perw, D), dtype)
                     for _ in range(n))

    def body(table_ref, *rest):
        idx_refs = rest[:n]
        out_refs = rest[n:2 * n]
        idx_v, rows_v, semg, semw = rest[2 * n:]
        w = lax.axis_index("s") * SC_CORES + lax.axis_index("c")
        base = w * perw
        for t in range(n):
            pltpu.sync_copy(idx_refs[t].at[w], idx_v)

            def group(g, _):
                hg = []
                for b in range(nbuf):
                    j = g * nbuf + b
                    hg.append(pltpu.async_copy(
                        table_ref.at[idx_v.at[j]], rows_v.at[b], semg))
                hw = []
                for b in range(nbuf):
                    j = g * nbuf + b
                    hg[b].wait()
                    hw.append(pltpu.async_copy(
                        rows_v.at[b],
                        out_refs[t].at[pl.ds(base + j * CH, CH)], semw))
                for h in hw:
                    h.wait()
                return 0

            lax.fori_loop(0, ngroups, group, 0)

    f = pl.kernel(
        body, out_type=out_type, mesh=_mesh(),
        scratch_types=[
            pltpu.VMEM((nch, CH), jnp.int32),
            pltpu.VMEM((nbuf, CH, D), dtype),
            pltpu.SemaphoreType.DMA,
            pltpu.SemaphoreType.DMA,
        ])
    return list(f(table, *idxs))


def _sc_scatter(vals, ridx, C, nbuf=2):
    ngroups = ECH // nbuf
    ntail = ECH % nbuf
    rpt = NPAD // SC_TILES
    zeros = jnp.zeros((CH, C), jnp.float32)
    out_type = jax.ShapeDtypeStruct((2 * NPAD, C), jnp.float32)

    def body(vals_ref, ridx_ref, z_ref, out01, idx_v, rows_v, acc,
             semg, sems):
        c = lax.axis_index("c")
        s = lax.axis_index("s")
        w = s * SC_CORES + c
        pltpu.sync_copy(z_ref, rows_v.at[0])
        for q in range(rpt // CH):
            pltpu.sync_copy(rows_v.at[0], acc.at[pl.ds(s * rpt + q * CH, CH)])
        pltpu.sync_copy(ridx_ref.at[w], idx_v)
        plsc.subcore_barrier()
        base = w * ECH * CH

        def group(g, _):
            hg = []
            for b in range(nbuf):
                j = g * nbuf + b
                hg.append(pltpu.async_copy(
                    vals_ref.at[pl.ds(base + j * CH, CH)], rows_v.at[b], semg))
            hs = []
            for b in range(nbuf):
                j = g * nbuf + b
                hg[b].wait()
                hs.append(pltpu.async_copy(
                    rows_v.at[b], acc.at[idx_v.at[j]], sems, add=True))
            for h in hs:
                h.wait()
            return 0

        lax.fori_loop(0, ngroups, group, 0)
        for t in range(ntail):
            j = ngroups * nbuf + t
            pltpu.sync_copy(vals_ref.at[pl.ds(base + j * CH, CH)], rows_v.at[0])
            pltpu.async_copy(rows_v.at[0], acc.at[idx_v.at[j]], sems,
                             add=True).wait()
        plsc.subcore_barrier()
        obase = c * NPAD + s * rpt
        for q in range(rpt // CH):
            pltpu.sync_copy(acc.at[pl.ds(s * rpt + q * CH, CH)],
                            rows_v.at[q % nbuf])
            pltpu.sync_copy(rows_v.at[q % nbuf],
                            out01.at[pl.ds(obase + q * CH, CH)])

    f = pl.kernel(
        body, out_type=out_type, mesh=_mesh(),
        scratch_types=[
            pltpu.VMEM((ECH, CH), jnp.int32),
            pltpu.VMEM((nbuf, CH, C), jnp.float32),
            pltpu.VMEM_SHARED((NPAD, C), jnp.float32),
            pltpu.SemaphoreType.DMA,
            pltpu.SemaphoreType.DMA,
        ])
    return f(vals, ridx, zeros)


def _silu(x):
    return x * jax.nn.sigmoid(x)


def _wspec(a):
    return pl.BlockSpec(a.shape, lambda i: (0,) * a.ndim)


def _xspec(be, d):
    return pl.BlockSpec((be, d), lambda i: (i, 0))


def _envelope(r):
    x = jnp.clip(r * (1.0 / CUTOFF), 0.0, 1.0)
    x6 = x * x * x * x * x * x
    env = 1.0 - 28.0 * x6 + 48.0 * x6 * x - 21.0 * x6 * x * x
    return jnp.where(r < CUTOFF, env, 0.0)


def _edge_call(r1, fi, fj, wr, wfi, wfj, b1, w2, b2, be=1280):
    grid = (E // be,)

    def body(r_ref, fi_ref, fj_ref, wr_r, wfi_r, wfj_r, b1_r, w2_r, b2_r, out):
        r = r_ref[...]
        rs = jnp.maximum(r, 1e-2)
        freq = (lax.broadcasted_iota(jnp.int32, (1, NB), 1) + 1
                ).astype(jnp.float32) * (jnp.pi / CUTOFF)
        rbf = (jnp.sin(freq * rs) / rs).astype(jnp.bfloat16)
        bf = jnp.bfloat16
        h = (jnp.dot(rbf, wr_r[...], preferred_element_type=jnp.float32)
             + jnp.dot(fi_ref[...].astype(bf), wfi_r[...],
                       preferred_element_type=jnp.float32)
             + jnp.dot(fj_ref[...].astype(bf), wfj_r[...],
                       preferred_element_type=jnp.float32)
             + b1_r[...])
        h = _silu(h).astype(jnp.bfloat16)
        out[...] = (jnp.dot(h, w2_r[...], preferred_element_type=jnp.float32)
                    + b2_r[...]).astype(jnp.bfloat16)

    return pl.pallas_call(
        body,
        grid=grid,
        in_specs=[_xspec(be, 1), _xspec(be, NODE), _xspec(be, NODE),
                  _wspec(wr), _wspec(wfi), _wspec(wfj), _wspec(b1),
                  _wspec(w2), _wspec(b2)],
        out_specs=_xspec(be, NODE),
        out_shape=jax.ShapeDtypeStruct((E, NODE), jnp.bfloat16),
    )(r1, fi, fj, wr, wfi, wfj, b1, w2, b2)


def _eqmsg_call(r1, fi, fj, pr, ps, ed, ws, dout, be=1280):
    grid = (E // be,)
    has_an = pr is not None
    wfi, wfj, wan, wed, b1, w2, b2, w3, b3 = ws

    def body(*refs):
        if has_an:
            (r_ref, fi_ref, fj_ref, pr_ref, ps_ref, ed_ref,
             wfi_r, wfj_r, wan_r, wed_r, b1_r, w2_r, b2_r, w3_r, b3_r,
             out) = refs
        else:
            (r_ref, fi_ref, fj_ref, ed_ref,
             wfi_r, wfj_r, wed_r, b1_r, w2_r, b2_r, w3_r, b3_r, out) = refs
        bf = jnp.bfloat16
        h = (jnp.dot(fi_ref[...].astype(bf), wfi_r[...],
                     preferred_element_type=jnp.float32)
             + jnp.dot(fj_ref[...].astype(bf), wfj_r[...],
                       preferred_element_type=jnp.float32)
             + jnp.dot(ed_ref[...], wed_r[...],
                       preferred_element_type=jnp.float32)
             + b1_r[...])
        if has_an:
            h = h + jnp.dot((pr_ref[...] - ps_ref[...]).astype(bf), wan_r[...],
                            preferred_element_type=jnp.float32)
        h = _silu(h).astype(jnp.bfloat16)
        h = _silu(jnp.dot(h, w2_r[...], preferred_element_type=jnp.float32)
                  + b2_r[...]).astype(jnp.bfloat16)
        o = jnp.dot(h, w3_r[...], preferred_element_type=jnp.float32) + b3_r[...]
        out[...] = o * _envelope(r_ref[...])

    xs = [r1, fi, fj] + ([pr, ps] if has_an else []) + [ed]
    wlist = [wfi, wfj] + ([wan] if has_an else []) + [wed, b1, w2, b2, w3, b3]
    in_specs = ([_xspec(be, 1), _xspec(be, NODE), _xspec(be, NODE)]
                + ([_xspec(be, CP), _xspec(be, CP)] if has_an else [])
                + [_xspec(be, NODE)]
                + [_wspec(w) for w in wlist])
    return pl.pallas_call(
        body,
        grid=grid,
        in_specs=in_specs,
        out_specs=_xspec(be, dout),
        out_shape=jax.ShapeDtypeStruct((E, dout), jnp.float32),
    )(*(xs + wlist))


def _halfspec(be, d):
    return pl.BlockSpec((be, d), lambda i: (i + NPAD // be, 0))


def _combine_call(p01, d, be=1024):
    grid = (NPAD // be,)

    def body(a, b, out):
        out[...] = a[...] + b[...]

    return pl.pallas_call(
        body, grid=grid,
        in_specs=[_xspec(be, d), _halfspec(be, d)],
        out_specs=_xspec(be, d),
        out_shape=jax.ShapeDtypeStruct((NPAD, d), jnp.float32),
    )(p01, p01)


def _upd_call(nodes, a01, ws, be=1024):
    grid = (NPAD // be,)
    wn, wa, b1, w2, b2, w3, b3 = ws

    def body(n_ref, a0_ref, a1_ref, wn_r, wa_r, b1_r, w2_r, b2_r, w3_r, b3_r,
             out):
        nd = n_ref[...]
        agg = a0_ref[...] + a1_ref[...]
        h = (jnp.dot(nd, wn_r[...], preferred_element_type=jnp.float32)
             + jnp.dot(agg, wa_r[...], preferred_element_type=jnp.float32)
             + b1_r[...])
        h = _silu(h)
        h = _silu(jnp.dot(h, w2_r[...], preferred_element_type=jnp.float32)
                  + b2_r[...])
        out[...] = nd + jnp.dot(h, w3_r[...],
                                preferred_element_type=jnp.float32) + b3_r[...]

    return pl.pallas_call(
        body, grid=grid,
        in_specs=[_xspec(be, NODE), _xspec(be, NODE), _halfspec(be, NODE)]
        + [_wspec(w) for w in ws],
        out_specs=_xspec(be, NODE),
        out_shape=jax.ShapeDtypeStruct((NPAD, NODE), jnp.float32),
    )(nodes, a01, a01, *ws)


def _mono_call(nodes, w1, b1, w2, b2, be=1024):
    grid = (NPAD // be,)

    def body(n_ref, w1_r, b1_r, w2_r, b2_r, out):
        h = _silu(jnp.dot(n_ref[...], w1_r[...],
                          preferred_element_type=jnp.float32) + b1_r[...])
        out[...] = (jnp.dot(h, w2_r[...], preferred_element_type=jnp.float32)
                    + b2_r[...]) * (1.0 / KEPS)

    return pl.pallas_call(
        body, grid=grid,
        in_specs=[_xspec(be, NODE), _wspec(w1), _wspec(b1), _wspec(w2),
                  _wspec(b2)],
        out_specs=_xspec(be, 1),
        out_shape=jax.ShapeDtypeStruct((NPAD, 1), jnp.float32),
    )(nodes, w1, b1, w2, b2)


def _row(b):
    return b.reshape(1, -1)


def _pad_cols(w, width):
    return jnp.pad(w, ((0, 0), (0, width - w.shape[1])))


def _pad_rows(w, height):
    return jnp.pad(w, ((0, height - w.shape[0]), (0, 0)))


def kernel(Z, R1, senders, receivers, params):
    Zp = jnp.pad(Z.astype(jnp.int32), (0, NPAD - N)).reshape(NW, ZCH, CH)
    send_r = senders.astype(jnp.int32).reshape(NW, ECH, CH)
    recv_r = receivers.astype(jnp.int32).reshape(NW, ECH, CH)
    r1c = R1.reshape(E, 1)

    bf = jnp.bfloat16

    nodes = _sc_gather(params["emb"], [Zp], D=NODE, nch=ZCH, nbuf=ZCH)[0]

    fi, fj = _sc_gather(nodes, [send_r, recv_r], D=NODE, nch=ECH, nbuf=5)

    (w1e, b1e), (w2e, b2e) = params["edge"]
    edges = _edge_call(r1c, fi, fj,
                       w1e[:NB].astype(bf), w1e[NB:NB + NODE].astype(bf),
                       w1e[NB + NODE:].astype(bf),
                       _row(b1e), w2e.astype(bf), _row(b2e))

    pr = ps = None
    for s, lp in enumerate(params["steps"]):
        (w1, b1), (w2, b2), (w3, b3) = lp["eq"]
        if s == 0:
            ws = (w1[:NODE].astype(bf), w1[NODE:2 * NODE].astype(bf), None,
                  w1[2 * NODE:].astype(bf),
                  _row(b1), w2.astype(bf), _row(b2),
                  _pad_cols(w3, CP).astype(bf),
                  _row(jnp.pad(b3, (0, CP - NC9))))
        else:
            ws = (w1[:NODE].astype(bf), w1[NODE:2 * NODE].astype(bf),
                  _pad_rows(w1[2 * NODE:2 * NODE + NC9], CP).astype(bf),
                  w1[2 * NODE + NC9:].astype(bf),
                  _row(b1), w2.astype(bf), _row(b2),
                  _pad_cols(w3, CP).astype(bf),
                  _row(jnp.pad(b3, (0, CP - NC9))))
        coeffs = _eqmsg_call(r1c, fi, fj, pr, ps, edges, ws, dout=CP)

        poles = _combine_call(_sc_scatter(coeffs, recv_r, C=CP), d=CP)

        pr, ps = _sc_gather(poles, [recv_r, send_r], D=CP, nch=ECH, nbuf=5)

        (w1, b1), (w2, b2), (w3, b3) = lp["msg"]
        ws = (w1[:NODE].astype(bf), w1[NODE:2 * NODE].astype(bf),
              _pad_rows(w1[2 * NODE:2 * NODE + NC9], CP).astype(bf),
              w1[2 * NODE + NC9:].astype(bf),
              _row(b1), w2.astype(bf), _row(b2), w3.astype(bf), _row(b3))
        msgs = _eqmsg_call(r1c, fi, fj, pr, ps, edges, ws, dout=NODE)

        a01 = _sc_scatter(msgs, recv_r, C=NODE)
        (w1, b1), (w2, b2), (w3, b3) = lp["upd"]
        nodes = _upd_call(nodes, a01,
                          (w1[:NODE], w1[NODE:], _row(b1), w2, _row(b2),
                           w3, _row(b3)))

        if s < len(params["steps"]) - 1:
            fi, fj = _sc_gather(nodes, [send_r, recv_r], D=NODE, nch=ECH,
                                nbuf=5)

    (w1m, b1m), (w2m, b2m) = params["mono"]
    monos = _mono_call(nodes, w1m, _row(b1m), w2m, _row(b2m))
    return jnp.concatenate([nodes[:N], monos[:N]], axis=-1)
